# baseline (speedup 1.0000x reference)
import functools

import jax
import jax.numpy as jnp
from jax import lax
from jax.experimental import pallas as pl
from jax.experimental.pallas import tpu as pltpu
from jax.experimental.pallas import tpu_sc as plsc

N_NODES = 10000
N_EDGES = 320000
D = 128
N_CLS = 64

NC = 2
NS = 16
NW = NC * NS
GROUP = 125
GROUPS_PER_TILE = 80
NPAD = 10112
ROWS_PER_TILE = NPAD // NS


CG = 8


def _sc_agg_body(h_hbm, src_hbm, dst_hbm, zeros_hbm, out_hbm,
                 agg_s, src_c, dst_c, rows0, rows1,
                 sem_i, sem_g0, sem_g1, sem_s0, sem_s1):
    c = lax.axis_index("c")
    s = lax.axis_index("s")
    w = c * NS + s
    nchunk = GROUPS_PER_TILE // CG
    gbase = w * GROUPS_PER_TILE
    rows = (rows0, rows1)
    sem_g = (sem_g0, sem_g1)
    sem_s = (sem_s0, sem_s1)

    def idx_fetch(ch, slot):
        base = gbase + ch * CG
        pltpu.async_copy(src_hbm.at[pl.ds(base, CG)], src_c.at[slot], sem_i)
        pltpu.async_copy(dst_hbm.at[pl.ds(base, CG)], dst_c.at[slot], sem_i)

    def idx_wait(slot):
        pltpu.make_async_copy(src_hbm.at[pl.ds(0, CG)], src_c.at[slot], sem_i).wait()
        pltpu.make_async_copy(dst_hbm.at[pl.ds(0, CG)], dst_c.at[slot], sem_i).wait()

    def gather(slot, j, b):
        pltpu.async_copy(h_hbm.at[src_c.at[slot, j]], rows[b], sem_g[b])

    def gather_wait(b):
        pltpu.make_async_copy(h_hbm.at[src_c.at[0, 0]], rows[b], sem_g[b]).wait()

    def scatter(slot, j, b):
        pltpu.async_copy(rows[b], agg_s.at[dst_c.at[slot, j]], sem_s[b], add=True)

    def scatter_wait(b):
        pltpu.make_async_copy(rows[b], agg_s.at[dst_c.at[0, 0]], sem_s[b]).wait()

    pltpu.sync_copy(zeros_hbm, agg_s.at[pl.ds(s * ROWS_PER_TILE, ROWS_PER_TILE)])
    idx_fetch(0, 0)
    plsc.subcore_barrier()

    def chunk_body(ch, carry):
        slot = lax.rem(ch, 2)
        idx_wait(slot)

        @pl.when(ch + 1 < nchunk)
        def _():
            idx_fetch(ch + 1, 1 - slot)

        gather(slot, 0, 0)
        gather(slot, 1, 1)
        for j in range(CG):
            b = j % 2
            gather_wait(b)
            scatter(slot, j, b)
            if j + 2 < CG:
                scatter_wait(b)
                gather(slot, j + 2, b)
        scatter_wait(0)
        scatter_wait(1)
        return carry

    lax.fori_loop(0, nchunk, chunk_body, 0)
    plsc.subcore_barrier()
    pltpu.sync_copy(agg_s.at[pl.ds(s * ROWS_PER_TILE, ROWS_PER_TILE)],
                    out_hbm.at[c, pl.ds(s * ROWS_PER_TILE, ROWS_PER_TILE)])


_sc_agg = functools.partial(
    pl.kernel,
    mesh=plsc.VectorSubcoreMesh(core_axis_name="c", subcore_axis_name="s"),
    out_type=jax.ShapeDtypeStruct((NC, NPAD, D), jnp.float32),
    scratch_types=[
        pltpu.VMEM_SHARED((NPAD, D), jnp.float32),
        pltpu.VMEM((2, CG, GROUP), jnp.int32),
        pltpu.VMEM((2, CG, GROUP), jnp.int32),
        pltpu.VMEM((GROUP, D), jnp.float32),
        pltpu.VMEM((GROUP, D), jnp.float32),
    ] + [pltpu.SemaphoreType.DMA] * 5,
)(_sc_agg_body)


_TC_ROWS = 1000


def _tc_layer_body(x_ref, agg_ref, w_ref, b_ref, o_ref):
    acc = x_ref[...] + agg_ref[0] + agg_ref[1]
    y = jnp.dot(acc, w_ref[...], preferred_element_type=jnp.float32) + b_ref[...]
    o_ref[...] = jnp.maximum(y, 0.0)


_tc_layer = pl.pallas_call(
    _tc_layer_body,
    grid=(N_NODES // _TC_ROWS,),
    in_specs=[
        pl.BlockSpec((_TC_ROWS, D), lambda i: (i, 0)),
        pl.BlockSpec((NC, _TC_ROWS, D), lambda i: (0, i, 0)),
        pl.BlockSpec((D, D), lambda i: (0, 0)),
        pl.BlockSpec((1, D), lambda i: (0, 0)),
    ],
    out_specs=pl.BlockSpec((_TC_ROWS, D), lambda i: (i, 0)),
    out_shape=jax.ShapeDtypeStruct((N_NODES, D), jnp.float32),
)


def _tc_final_body(x_ref, agg_ref, w1_ref, b1_ref, w2_ref, b2_ref, o_ref):
    acc = x_ref[...] + agg_ref[0] + agg_ref[1]
    h = jnp.dot(acc, w1_ref[...], preferred_element_type=jnp.float32) + b1_ref[...]
    h = jnp.maximum(h, 0.0)
    o_ref[...] = jnp.dot(h, w2_ref[...], preferred_element_type=jnp.float32) + b2_ref[...]


_tc_final = pl.pallas_call(
    _tc_final_body,
    grid=(N_NODES // _TC_ROWS,),
    in_specs=[
        pl.BlockSpec((_TC_ROWS, D), lambda i: (i, 0)),
        pl.BlockSpec((NC, _TC_ROWS, D), lambda i: (0, i, 0)),
        pl.BlockSpec((D, D), lambda i: (0, 0)),
        pl.BlockSpec((1, D), lambda i: (0, 0)),
        pl.BlockSpec((D, N_CLS), lambda i: (0, 0)),
        pl.BlockSpec((1, N_CLS), lambda i: (0, 0)),
    ],
    out_specs=pl.BlockSpec((_TC_ROWS, N_CLS), lambda i: (i, 0)),
    out_shape=jax.ShapeDtypeStruct((N_NODES, N_CLS), jnp.float32),
)


def kernel(x, edge_index, W0, b0, W1, b1, W2, b2):
    src2d = edge_index[0].reshape(-1, GROUP)
    dst2d = edge_index[1].reshape(-1, GROUP)
    zeros = jnp.zeros((ROWS_PER_TILE, D), jnp.float32)

    agg1 = _sc_agg(x, src2d, dst2d, zeros)
    h1 = _tc_layer(x, agg1, W0, b0.reshape(1, D))
    agg2 = _sc_agg(h1, src2d, dst2d, zeros)
    return _tc_final(h1, agg2, W1, b1.reshape(1, D), W2, b2.reshape(1, N_CLS))


# speedup vs baseline: 1.0438x; 1.0438x over previous
import functools

import jax
import jax.numpy as jnp
from jax import lax
from jax.experimental import pallas as pl
from jax.experimental.pallas import tpu as pltpu
from jax.experimental.pallas import tpu_sc as plsc

N_NODES = 10000
N_EDGES = 320000
D = 128
N_CLS = 64

NC = 2
NS = 16
NW = NC * NS
GROUP = 125
GROUPS_PER_TILE = 80
NPAD = 10112
ROWS_PER_TILE = NPAD // NS


CG = 16


def _sc_agg_body(h_hbm, src_hbm, dst_hbm, zeros_hbm, out_hbm,
                 agg_s, src_c, dst_c, rows0, rows1,
                 sem_i, sem_g0, sem_g1, sem_s0, sem_s1):
    c = lax.axis_index("c")
    s = lax.axis_index("s")
    w = c * NS + s
    nchunk = GROUPS_PER_TILE // CG
    gbase = w * GROUPS_PER_TILE
    rows = (rows0, rows1)
    sem_g = (sem_g0, sem_g1)
    sem_s = (sem_s0, sem_s1)

    def idx_fetch(ch, slot):
        base = gbase + ch * CG
        pltpu.async_copy(src_hbm.at[pl.ds(base, CG)], src_c.at[slot], sem_i)
        pltpu.async_copy(dst_hbm.at[pl.ds(base, CG)], dst_c.at[slot], sem_i)

    def idx_wait(slot):
        pltpu.make_async_copy(src_hbm.at[pl.ds(0, CG)], src_c.at[slot], sem_i).wait()
        pltpu.make_async_copy(dst_hbm.at[pl.ds(0, CG)], dst_c.at[slot], sem_i).wait()

    def gather(slot, j, b):
        pltpu.async_copy(h_hbm.at[src_c.at[slot, j]], rows[b], sem_g[b])

    def gather_wait(b):
        pltpu.make_async_copy(h_hbm.at[src_c.at[0, 0]], rows[b], sem_g[b]).wait()

    def scatter(slot, j, b):
        pltpu.async_copy(rows[b], agg_s.at[dst_c.at[slot, j]], sem_s[b], add=True)

    def scatter_wait(b):
        pltpu.make_async_copy(rows[b], agg_s.at[dst_c.at[0, 0]], sem_s[b]).wait()

    pltpu.sync_copy(zeros_hbm, agg_s.at[pl.ds(s * ROWS_PER_TILE, ROWS_PER_TILE)])
    idx_fetch(0, 0)
    plsc.subcore_barrier()

    def chunk_body(ch, carry):
        slot = lax.rem(ch, 2)
        idx_wait(slot)

        @pl.when(ch + 1 < nchunk)
        def _():
            idx_fetch(ch + 1, 1 - slot)

        gather(slot, 0, 0)
        gather(slot, 1, 1)
        for j in range(CG):
            b = j % 2
            gather_wait(b)
            scatter(slot, j, b)
            if j + 2 < CG:
                scatter_wait(b)
                gather(slot, j + 2, b)
        scatter_wait(0)
        scatter_wait(1)
        return carry

    lax.fori_loop(0, nchunk, chunk_body, 0)
    plsc.subcore_barrier()
    pltpu.sync_copy(agg_s.at[pl.ds(s * ROWS_PER_TILE, ROWS_PER_TILE)],
                    out_hbm.at[c, pl.ds(s * ROWS_PER_TILE, ROWS_PER_TILE)])


_sc_agg = functools.partial(
    pl.kernel,
    mesh=plsc.VectorSubcoreMesh(core_axis_name="c", subcore_axis_name="s"),
    out_type=jax.ShapeDtypeStruct((NC, NPAD, D), jnp.float32),
    scratch_types=[
        pltpu.VMEM_SHARED((NPAD, D), jnp.float32),
        pltpu.VMEM((2, CG, GROUP), jnp.int32),
        pltpu.VMEM((2, CG, GROUP), jnp.int32),
        pltpu.VMEM((GROUP, D), jnp.float32),
        pltpu.VMEM((GROUP, D), jnp.float32),
    ] + [pltpu.SemaphoreType.DMA] * 5,
)(_sc_agg_body)


_TC_ROWS = 1000


def _tc_layer_body(x_ref, agg_ref, w_ref, b_ref, o_ref):
    acc = x_ref[...] + agg_ref[0] + agg_ref[1]
    y = jnp.dot(acc, w_ref[...], preferred_element_type=jnp.float32) + b_ref[...]
    o_ref[...] = jnp.maximum(y, 0.0)


_tc_layer = pl.pallas_call(
    _tc_layer_body,
    grid=(N_NODES // _TC_ROWS,),
    in_specs=[
        pl.BlockSpec((_TC_ROWS, D), lambda i: (i, 0)),
        pl.BlockSpec((NC, _TC_ROWS, D), lambda i: (0, i, 0)),
        pl.BlockSpec((D, D), lambda i: (0, 0)),
        pl.BlockSpec((1, D), lambda i: (0, 0)),
    ],
    out_specs=pl.BlockSpec((_TC_ROWS, D), lambda i: (i, 0)),
    out_shape=jax.ShapeDtypeStruct((N_NODES, D), jnp.float32),
)


def _tc_final_body(x_ref, agg_ref, w1_ref, b1_ref, w2_ref, b2_ref, o_ref):
    acc = x_ref[...] + agg_ref[0] + agg_ref[1]
    h = jnp.dot(acc, w1_ref[...], preferred_element_type=jnp.float32) + b1_ref[...]
    h = jnp.maximum(h, 0.0)
    o_ref[...] = jnp.dot(h, w2_ref[...], preferred_element_type=jnp.float32) + b2_ref[...]


_tc_final = pl.pallas_call(
    _tc_final_body,
    grid=(N_NODES // _TC_ROWS,),
    in_specs=[
        pl.BlockSpec((_TC_ROWS, D), lambda i: (i, 0)),
        pl.BlockSpec((NC, _TC_ROWS, D), lambda i: (0, i, 0)),
        pl.BlockSpec((D, D), lambda i: (0, 0)),
        pl.BlockSpec((1, D), lambda i: (0, 0)),
        pl.BlockSpec((D, N_CLS), lambda i: (0, 0)),
        pl.BlockSpec((1, N_CLS), lambda i: (0, 0)),
    ],
    out_specs=pl.BlockSpec((_TC_ROWS, N_CLS), lambda i: (i, 0)),
    out_shape=jax.ShapeDtypeStruct((N_NODES, N_CLS), jnp.float32),
)


def kernel(x, edge_index, W0, b0, W1, b1, W2, b2):
    src2d = edge_index[0].reshape(-1, GROUP)
    dst2d = edge_index[1].reshape(-1, GROUP)
    zeros = jnp.zeros((ROWS_PER_TILE, D), jnp.float32)

    agg1 = _sc_agg(x, src2d, dst2d, zeros)
    h1 = _tc_layer(x, agg1, W0, b0.reshape(1, D))
    agg2 = _sc_agg(h1, src2d, dst2d, zeros)
    return _tc_final(h1, agg2, W1, b1.reshape(1, D), W2, b2.reshape(1, N_CLS))


# speedup vs baseline: 1.0743x; 1.0292x over previous
import functools

import jax
import jax.numpy as jnp
from jax import lax
from jax.experimental import pallas as pl
from jax.experimental.pallas import tpu as pltpu
from jax.experimental.pallas import tpu_sc as plsc

N_NODES = 10000
N_EDGES = 320000
D = 128
N_CLS = 64

NC = 2
NS = 16
NW = NC * NS
GROUP = 125
GROUPS_PER_TILE = 80
NPAD = 10112
ROWS_PER_TILE = NPAD // NS


CG = 16


def _sc_agg_body(h_hbm, src_hbm, dst_hbm, zeros_hbm, out_hbm,
                 agg_s, src_c, dst_c, rows0, rows1,
                 sem_i, sem_g0, sem_g1, sem_s0, sem_s1):
    c = lax.axis_index("c")
    s = lax.axis_index("s")
    w = c * NS + s
    nchunk = GROUPS_PER_TILE // CG
    gbase = w * GROUPS_PER_TILE
    rows = (rows0, rows1)
    sem_g = (sem_g0, sem_g1)
    sem_s = (sem_s0, sem_s1)

    def idx_fetch(ch, slot):
        base = gbase + ch * CG
        pltpu.async_copy(src_hbm.at[pl.ds(base, CG)], src_c.at[slot], sem_i)
        pltpu.async_copy(dst_hbm.at[pl.ds(base, CG)], dst_c.at[slot], sem_i)

    def idx_wait(slot):
        pltpu.make_async_copy(src_hbm.at[pl.ds(0, CG)], src_c.at[slot], sem_i).wait()
        pltpu.make_async_copy(dst_hbm.at[pl.ds(0, CG)], dst_c.at[slot], sem_i).wait()

    def gather(slot, j, b):
        pltpu.async_copy(h_hbm.at[src_c.at[slot, j]], rows[b], sem_g[b])

    def gather_wait(b):
        pltpu.make_async_copy(h_hbm.at[src_c.at[0, 0]], rows[b], sem_g[b]).wait()

    def scatter(slot, j, b):
        pltpu.async_copy(rows[b], agg_s.at[dst_c.at[slot, j]], sem_s[b], add=True)

    def scatter_wait(b):
        pltpu.make_async_copy(rows[b], agg_s.at[dst_c.at[0, 0]], sem_s[b]).wait()

    pltpu.sync_copy(zeros_hbm, agg_s.at[pl.ds(s * ROWS_PER_TILE, ROWS_PER_TILE)])
    idx_fetch(0, 0)
    plsc.subcore_barrier()
    idx_wait(0)
    gather(0, 0, 0)
    gather(0, 1, 1)

    def chunk_body(ch, carry):
        slot = lax.rem(ch, 2)

        @pl.when(ch + 1 < nchunk)
        def _():
            idx_fetch(ch + 1, 1 - slot)

        for j in range(CG):
            b = j % 2
            gather_wait(b)
            scatter(slot, j, b)
            if j + 2 < CG:
                scatter_wait(b)
                gather(slot, j + 2, b)

        @pl.when(ch + 1 < nchunk)
        def _():
            idx_wait(1 - slot)
            scatter_wait(0)
            gather(1 - slot, 0, 0)
            scatter_wait(1)
            gather(1 - slot, 1, 1)

        @pl.when(ch + 1 >= nchunk)
        def _():
            scatter_wait(0)
            scatter_wait(1)

        return carry

    lax.fori_loop(0, nchunk, chunk_body, 0)
    plsc.subcore_barrier()
    pltpu.sync_copy(agg_s.at[pl.ds(s * ROWS_PER_TILE, ROWS_PER_TILE)],
                    out_hbm.at[c, pl.ds(s * ROWS_PER_TILE, ROWS_PER_TILE)])


_sc_agg = functools.partial(
    pl.kernel,
    mesh=plsc.VectorSubcoreMesh(core_axis_name="c", subcore_axis_name="s"),
    out_type=jax.ShapeDtypeStruct((NC, NPAD, D), jnp.float32),
    scratch_types=[
        pltpu.VMEM_SHARED((NPAD, D), jnp.float32),
        pltpu.VMEM((2, CG, GROUP), jnp.int32),
        pltpu.VMEM((2, CG, GROUP), jnp.int32),
        pltpu.VMEM((GROUP, D), jnp.float32),
        pltpu.VMEM((GROUP, D), jnp.float32),
    ] + [pltpu.SemaphoreType.DMA] * 5,
)(_sc_agg_body)


_TC_ROWS = 1000


def _tc_layer_body(x_ref, agg_ref, w_ref, b_ref, o_ref):
    acc = x_ref[...] + agg_ref[0] + agg_ref[1]
    y = jnp.dot(acc, w_ref[...], preferred_element_type=jnp.float32) + b_ref[...]
    o_ref[...] = jnp.maximum(y, 0.0)


_tc_layer = pl.pallas_call(
    _tc_layer_body,
    grid=(N_NODES // _TC_ROWS,),
    in_specs=[
        pl.BlockSpec((_TC_ROWS, D), lambda i: (i, 0)),
        pl.BlockSpec((NC, _TC_ROWS, D), lambda i: (0, i, 0)),
        pl.BlockSpec((D, D), lambda i: (0, 0)),
        pl.BlockSpec((1, D), lambda i: (0, 0)),
    ],
    out_specs=pl.BlockSpec((_TC_ROWS, D), lambda i: (i, 0)),
    out_shape=jax.ShapeDtypeStruct((N_NODES, D), jnp.float32),
)


def _tc_final_body(x_ref, agg_ref, w1_ref, b1_ref, w2_ref, b2_ref, o_ref):
    acc = x_ref[...] + agg_ref[0] + agg_ref[1]
    h = jnp.dot(acc, w1_ref[...], preferred_element_type=jnp.float32) + b1_ref[...]
    h = jnp.maximum(h, 0.0)
    o_ref[...] = jnp.dot(h, w2_ref[...], preferred_element_type=jnp.float32) + b2_ref[...]


_tc_final = pl.pallas_call(
    _tc_final_body,
    grid=(N_NODES // _TC_ROWS,),
    in_specs=[
        pl.BlockSpec((_TC_ROWS, D), lambda i: (i, 0)),
        pl.BlockSpec((NC, _TC_ROWS, D), lambda i: (0, i, 0)),
        pl.BlockSpec((D, D), lambda i: (0, 0)),
        pl.BlockSpec((1, D), lambda i: (0, 0)),
        pl.BlockSpec((D, N_CLS), lambda i: (0, 0)),
        pl.BlockSpec((1, N_CLS), lambda i: (0, 0)),
    ],
    out_specs=pl.BlockSpec((_TC_ROWS, N_CLS), lambda i: (i, 0)),
    out_shape=jax.ShapeDtypeStruct((N_NODES, N_CLS), jnp.float32),
)


def kernel(x, edge_index, W0, b0, W1, b1, W2, b2):
    src2d = edge_index[0].reshape(-1, GROUP)
    dst2d = edge_index[1].reshape(-1, GROUP)
    zeros = jnp.zeros((ROWS_PER_TILE, D), jnp.float32)

    agg1 = _sc_agg(x, src2d, dst2d, zeros)
    h1 = _tc_layer(x, agg1, W0, b0.reshape(1, D))
    agg2 = _sc_agg(h1, src2d, dst2d, zeros)
    return _tc_final(h1, agg2, W1, b1.reshape(1, D), W2, b2.reshape(1, N_CLS))
